# Initial kernel scaffold; baseline (speedup 1.0000x reference)
import jax
import jax.numpy as jnp
from jax.experimental import pallas as pl


def kernel(inputs, input_lens, table):
    raise NotImplementedError("write your pallas kernel here")


# speedup vs baseline: 1.9016x; 1.9016x over previous
import functools

import jax
import jax.numpy as jnp
from jax import lax
from jax.experimental import pallas as pl
from jax.experimental.pallas import tpu as pltpu
from jax.experimental.pallas import tpu_sc as plsc

_BATCH = 4096
_HIST = 200
_DIM = 32


def _make_kernel(nc, ns, bpw):
    mesh = plsc.VectorSubcoreMesh(core_axis_name="c", subcore_axis_name="s")

    @functools.partial(
        pl.kernel,
        mesh=mesh,
        compiler_params=pltpu.CompilerParams(use_tc_tiling_on_sc=False),
        out_type=jax.ShapeDtypeStruct((_BATCH, _DIM), jnp.float32),
        scratch_types=[
            pltpu.VMEM((bpw * _HIST,), jnp.int32),
            pltpu.VMEM((bpw, 16), jnp.float32),
            pltpu.VMEM((_HIST, _DIM), jnp.float32),
            pltpu.VMEM((bpw, _DIM), jnp.float32),
            pltpu.SemaphoreType.DMA,
        ],
    )
    def k(idx_hbm, lens_hbm, table_hbm, out_hbm, idx_v, lens_v, buf, out_v, sem):
        wid = lax.axis_index("s") * nc + lax.axis_index("c")
        base = wid * bpw
        pltpu.sync_copy(idx_hbm.at[wid], idx_v)
        pltpu.sync_copy(lens_hbm.at[pl.ds(base, bpw)], lens_v)

        def row_body(j, carry):
            pltpu.async_copy(
                table_hbm.at[idx_v.at[pl.ds(j * _HIST, _HIST)]], buf, sem
            ).wait()

            def red(l, acc):
                a0, a1 = acc
                return (a0 + buf[l, pl.ds(0, 16)], a1 + buf[l, pl.ds(16, 16)])

            zero = jnp.zeros((16,), jnp.float32)
            a0, a1 = lax.fori_loop(0, _HIST, red, (zero, zero))
            lenv = lens_v[j, pl.ds(0, 16)]
            out_v[j, pl.ds(0, 16)] = a0 / lenv
            out_v[j, pl.ds(16, 16)] = a1 / lenv
            return carry

        lax.fori_loop(0, bpw, row_body, 0)
        pltpu.sync_copy(out_v, out_hbm.at[pl.ds(base, bpw)])

    return k


def kernel(inputs, input_lens, table):
    info = plsc.get_sparse_core_info()
    nc, ns = info.num_cores, info.num_subcores
    nw = nc * ns
    bpw = _BATCH // nw
    idx = inputs.reshape(nw, bpw * _HIST)
    lens = jnp.broadcast_to(input_lens.reshape(_BATCH, 1), (_BATCH, 16))
    k = _make_kernel(nc, ns, bpw)
    return k(idx, lens, table)


# speedup vs baseline: 2.4332x; 1.2795x over previous
import functools

import jax
import jax.numpy as jnp
from jax import lax
from jax.experimental import pallas as pl
from jax.experimental.pallas import tpu as pltpu
from jax.experimental.pallas import tpu_sc as plsc

_BATCH = 4096
_HIST = 200
_DIM = 32
_CHUNK = 4
_ROWS = _CHUNK * _HIST


def _make_kernel(nc, ns, bpw):
    mesh = plsc.VectorSubcoreMesh(core_axis_name="c", subcore_axis_name="s")
    n_chunks = bpw // _CHUNK

    @functools.partial(
        pl.kernel,
        mesh=mesh,
        compiler_params=pltpu.CompilerParams(use_tc_tiling_on_sc=False),
        out_type=jax.ShapeDtypeStruct((_BATCH, _DIM), jnp.float32),
        scratch_types=[
            pltpu.VMEM((bpw * _HIST,), jnp.int32),
            pltpu.VMEM((bpw, 16), jnp.float32),
            pltpu.VMEM((_ROWS, _DIM), jnp.float32),
            pltpu.VMEM((_ROWS, _DIM), jnp.float32),
            pltpu.VMEM((bpw, _DIM), jnp.float32),
            pltpu.SemaphoreType.DMA,
            pltpu.SemaphoreType.DMA,
        ],
    )
    def k(idx_hbm, lens_hbm, table_hbm, out_hbm,
          idx_v, lens_v, buf0, buf1, out_v, sem0, sem1):
        wid = lax.axis_index("s") * nc + lax.axis_index("c")
        base = wid * bpw
        pltpu.sync_copy(idx_hbm.at[wid], idx_v)
        pltpu.sync_copy(lens_hbm.at[pl.ds(base, bpw)], lens_v)

        def gather(cc, buf, sem):
            return pltpu.async_copy(
                table_hbm.at[idx_v.at[pl.ds(cc * _ROWS, _ROWS)]], buf, sem)

        gather(0, buf0, sem0)

        def super_body(g, carry):
            for b in range(2):
                cc = 2 * g + b
                bufc, semc = (buf0, sem0) if b == 0 else (buf1, sem1)
                bufn, semn = (buf1, sem1) if b == 0 else (buf0, sem0)

                @pl.when(cc + 1 < n_chunks)
                def _():
                    gather(cc + 1, bufn, semn)

                pltpu.make_async_copy(
                    table_hbm.at[idx_v.at[pl.ds(cc * _ROWS, _ROWS)]],
                    bufc, semc).wait()

                for jj in range(_CHUNK):
                    rbase = jj * _HIST

                    def red(l, acc):
                        accs = list(acc)
                        r0 = rbase + l * 8
                        for t in range(8):
                            p = t % 4
                            accs[2 * p] = accs[2 * p] + bufc[r0 + t, pl.ds(0, 16)]
                            accs[2 * p + 1] = (
                                accs[2 * p + 1] + bufc[r0 + t, pl.ds(16, 16)])
                        return tuple(accs)

                    zero = jnp.zeros((16,), jnp.float32)
                    accs = lax.fori_loop(0, _HIST // 8, red, (zero,) * 8)
                    a0 = (accs[0] + accs[2]) + (accs[4] + accs[6])
                    a1 = (accs[1] + accs[3]) + (accs[5] + accs[7])
                    j = cc * _CHUNK + jj
                    lenv = lens_v[j, pl.ds(0, 16)]
                    out_v[j, pl.ds(0, 16)] = a0 / lenv
                    out_v[j, pl.ds(16, 16)] = a1 / lenv
            return carry

        lax.fori_loop(0, n_chunks // 2, super_body, 0)
        pltpu.sync_copy(out_v, out_hbm.at[pl.ds(base, bpw)])

    return k


def kernel(inputs, input_lens, table):
    info = plsc.get_sparse_core_info()
    nc, ns = info.num_cores, info.num_subcores
    nw = nc * ns
    bpw = _BATCH // nw
    idx = inputs.reshape(nw, bpw * _HIST)
    lens = jnp.broadcast_to(input_lens.reshape(_BATCH, 1), (_BATCH, 16))
    k = _make_kernel(nc, ns, bpw)
    return k(idx, lens, table)
